# Initial kernel scaffold; baseline (speedup 1.0000x reference)
import jax
import jax.numpy as jnp
from jax.experimental import pallas as pl


def kernel(token_indices, table, pe):
    raise NotImplementedError("write your pallas kernel here")


# speedup vs baseline: 5.5406x; 5.5406x over previous
import functools
import math

import jax
import jax.numpy as jnp
from jax import lax
from jax.experimental import pallas as pl
from jax.experimental.pallas import tpu as pltpu
from jax.experimental.pallas import tpu_sc as plsc

D = 128
SEQ = 200
BATCH = 4096
VOCAB = 4
NC, NS = 2, 16
NW = NC * NS
ROWS = BATCH * SEQ
RPW = ROWS // NW
CHUNK = 128
NCHUNK = RPW // CHUNK
GROUP = 4
NSTEP = NCHUNK // GROUP


def _prep_body(tok_ref, table_ref, pe_ref, comb_ref, gidx_ref):
    scale = jnp.float32(math.sqrt(float(D)))
    pe = pe_ref[...]
    for v in range(VOCAB):
        comb_ref[:, v, :] = pe + table_ref[v, :][None, :] * scale
    pos = lax.broadcasted_iota(jnp.int32, (BATCH, SEQ), 1)
    gidx_ref[...] = tok_ref[...] + VOCAB * pos


def _sc_body(comb_hbm, gidx_hbm, out_hbm, idx_v, rows_v, gsem, ssem):
    wid = lax.axis_index("s") * NC + lax.axis_index("c")
    base = wid * RPW
    pltpu.sync_copy(gidx_hbm.at[wid], idx_v)

    def step(s, carry):
        copies = [
            pltpu.async_copy(
                comb_hbm.at[idx_v.at[s * GROUP + g]],
                rows_v.at[pl.ds(g * CHUNK, CHUNK)],
                gsem,
            )
            for g in range(GROUP)
        ]
        for cp in copies:
            cp.wait()
        pltpu.async_copy(
            rows_v, out_hbm.at[pl.ds(base + s * GROUP * CHUNK, GROUP * CHUNK)], ssem
        ).wait()
        return carry

    lax.fori_loop(0, NSTEP, step, 0)


def kernel(token_indices, table, pe):
    comb, gidx = pl.pallas_call(
        _prep_body,
        out_shape=(
            jax.ShapeDtypeStruct((SEQ, VOCAB, D), jnp.float32),
            jax.ShapeDtypeStruct((BATCH, SEQ), jnp.int32),
        ),
    )(token_indices, table, pe[:SEQ])

    comb = comb.reshape(SEQ * VOCAB, D)
    gidx3 = gidx.reshape(NW, RPW // CHUNK, CHUNK)

    sc = pl.kernel(
        _sc_body,
        out_type=jax.ShapeDtypeStruct((ROWS, D), jnp.float32),
        mesh=plsc.VectorSubcoreMesh(
            core_axis_name="c", subcore_axis_name="s", num_cores=NC, num_subcores=NS
        ),
        scratch_types=[
            pltpu.VMEM((RPW // CHUNK, CHUNK), jnp.int32),
            pltpu.VMEM((GROUP * CHUNK, D), jnp.float32),
            pltpu.SemaphoreType.DMA,
            pltpu.SemaphoreType.DMA,
        ],
    )
    out = sc(comb, gidx3)
    return out.reshape(BATCH, SEQ, D)


# speedup vs baseline: 14.4994x; 2.6170x over previous
import functools
import math

import jax
import jax.numpy as jnp
from jax import lax
from jax.experimental import pallas as pl
from jax.experimental.pallas import tpu as pltpu
from jax.experimental.pallas import tpu_sc as plsc

D = 128
SEQ = 200
BATCH = 4096
VOCAB = 4
NC, NS = 2, 16
NW = NC * NS
ROWS = BATCH * SEQ
RPW = ROWS // NW
CHUNK = 128
NCHUNK = RPW // CHUNK
GROUP = 2
NSTEP = NCHUNK // GROUP
NPAIR = NSTEP // 2
TROWS = SEQ * VOCAB


def _prep_body(tok_ref, table_ref, pe_ref, comb_ref, gidx_ref):
    scale = jnp.float32(math.sqrt(float(D)))
    pe = pe_ref[...]
    for v in range(VOCAB):
        comb_ref[:, v, :] = pe + table_ref[v, :][None, :] * scale
    pos = lax.broadcasted_iota(jnp.int32, (BATCH, SEQ), 1)
    gidx_ref[...] = tok_ref[...] + VOCAB * pos


def _sc_body(comb_hbm, gidx_hbm, out_hbm, idx_v, buf_a, buf_b, comb_sh,
             gsem_a, gsem_b, ssem_a, ssem_b):
    cid = lax.axis_index("c")
    sid = lax.axis_index("s")
    wid = sid * NC + cid
    base = wid * RPW

    @pl.when(sid == 0)
    def _stage():
        for h in range(4):
            sl = pl.ds(h * 200, 200)
            pltpu.sync_copy(comb_hbm.at[sl], buf_a.at[pl.ds(0, 200)])
            pltpu.sync_copy(buf_a.at[pl.ds(0, 200)], comb_sh.at[sl])

    plsc.subcore_barrier()

    pltpu.sync_copy(gidx_hbm.at[wid], idx_v)

    def gather(s, buf, sem, issue):
        for g in range(GROUP):
            cp = pltpu.make_async_copy(
                comb_sh.at[idx_v.at[s * GROUP + g]],
                buf.at[pl.ds(g * CHUNK, CHUNK)],
                sem,
            )
            if issue:
                cp.start()
            else:
                cp.wait()

    def scatter(s, buf, sem, issue):
        cp = pltpu.make_async_copy(
            buf, out_hbm.at[pl.ds(base + s * GROUP * CHUNK, GROUP * CHUNK)], sem
        )
        if issue:
            cp.start()
        else:
            cp.wait()

    gather(0, buf_a, gsem_a, True)

    def body(k, carry):
        s0 = 2 * k

        @pl.when(k > 0)
        def _():
            scatter(s0 - 1, buf_b, ssem_b, False)

        gather(s0 + 1, buf_b, gsem_b, True)
        gather(s0, buf_a, gsem_a, False)
        scatter(s0, buf_a, ssem_a, True)

        scatter(s0, buf_a, ssem_a, False)

        @pl.when(k < NPAIR - 1)
        def _():
            gather(s0 + 2, buf_a, gsem_a, True)

        gather(s0 + 1, buf_b, gsem_b, False)
        scatter(s0 + 1, buf_b, ssem_b, True)
        return carry

    lax.fori_loop(0, NPAIR, body, 0)
    scatter(NSTEP - 1, buf_b, ssem_b, False)


def kernel(token_indices, table, pe):
    comb, gidx = pl.pallas_call(
        _prep_body,
        out_shape=(
            jax.ShapeDtypeStruct((SEQ, VOCAB, D), jnp.float32),
            jax.ShapeDtypeStruct((BATCH, SEQ), jnp.int32),
        ),
    )(token_indices, table, pe[:SEQ])

    comb = comb.reshape(TROWS, D)
    gidx3 = gidx.reshape(NW, NCHUNK, CHUNK)

    sc = pl.kernel(
        _sc_body,
        out_type=jax.ShapeDtypeStruct((ROWS, D), jnp.float32),
        mesh=plsc.VectorSubcoreMesh(
            core_axis_name="c", subcore_axis_name="s", num_cores=NC, num_subcores=NS
        ),
        scratch_types=[
            pltpu.VMEM((NCHUNK, CHUNK), jnp.int32),
            pltpu.VMEM((GROUP * CHUNK, D), jnp.float32),
            pltpu.VMEM((GROUP * CHUNK, D), jnp.float32),
            pltpu.VMEM_SHARED((TROWS, D), jnp.float32),
            pltpu.SemaphoreType.DMA,
            pltpu.SemaphoreType.DMA,
            pltpu.SemaphoreType.DMA,
            pltpu.SemaphoreType.DMA,
        ],
    )
    out = sc(comb, gidx3)
    return out.reshape(BATCH, SEQ, D)


# speedup vs baseline: 15.0930x; 1.0409x over previous
import functools
import math

import jax
import jax.numpy as jnp
from jax import lax
from jax.experimental import pallas as pl
from jax.experimental.pallas import tpu as pltpu
from jax.experimental.pallas import tpu_sc as plsc

D = 128
SEQ = 200
BATCH = 4096
VOCAB = 4
NC, NS = 2, 16
NW = NC * NS
ROWS = BATCH * SEQ
RPW = ROWS // NW
CHUNK = 128
NCHUNK = RPW // CHUNK
GROUP = 2
NSTEP = NCHUNK // GROUP
NPAIR = NSTEP // 2
TROWS = SEQ * VOCAB


def _prep_body(tok_ref, table_ref, pe_ref, comb_ref, gidx_ref):
    scale = jnp.float32(math.sqrt(float(D)))
    pe = pe_ref[...]
    for v in range(VOCAB):
        comb_ref[:, v, :] = pe + table_ref[v, :][None, :] * scale
    pos = lax.broadcasted_iota(jnp.int32, (BATCH, SEQ), 1)
    gidx_ref[...] = tok_ref[...] + VOCAB * pos


def _sc_body(comb_hbm, gidx_hbm, out_hbm, idx_v, buf_a, buf_b, comb_sh,
             gsem_a, gsem_b, ssem_a, ssem_b):
    cid = lax.axis_index("c")
    sid = lax.axis_index("s")
    wid = sid * NC + cid
    base = wid * RPW

    srows = 80

    @pl.when(sid < TROWS // srows)
    def _stage():
        off = pl.multiple_of(sid * srows, 8)
        pltpu.sync_copy(comb_hbm.at[pl.ds(off, srows)], buf_a.at[pl.ds(0, srows)])
        pltpu.sync_copy(buf_a.at[pl.ds(0, srows)], comb_sh.at[pl.ds(off, srows)])

    plsc.subcore_barrier()

    pltpu.sync_copy(gidx_hbm.at[wid], idx_v)

    def gather(s, buf, sem, issue):
        for g in range(GROUP):
            cp = pltpu.make_async_copy(
                comb_sh.at[idx_v.at[s * GROUP + g]],
                buf.at[pl.ds(g * CHUNK, CHUNK)],
                sem,
            )
            if issue:
                cp.start()
            else:
                cp.wait()

    def scatter(s, buf, sem, issue):
        cp = pltpu.make_async_copy(
            buf, out_hbm.at[pl.ds(base + s * GROUP * CHUNK, GROUP * CHUNK)], sem
        )
        if issue:
            cp.start()
        else:
            cp.wait()

    gather(0, buf_a, gsem_a, True)

    def body(k, carry):
        s0 = 2 * k

        @pl.when(k > 0)
        def _():
            scatter(s0 - 1, buf_b, ssem_b, False)

        gather(s0 + 1, buf_b, gsem_b, True)
        gather(s0, buf_a, gsem_a, False)
        scatter(s0, buf_a, ssem_a, True)

        scatter(s0, buf_a, ssem_a, False)

        @pl.when(k < NPAIR - 1)
        def _():
            gather(s0 + 2, buf_a, gsem_a, True)

        gather(s0 + 1, buf_b, gsem_b, False)
        scatter(s0 + 1, buf_b, ssem_b, True)
        return carry

    lax.fori_loop(0, NPAIR, body, 0)
    scatter(NSTEP - 1, buf_b, ssem_b, False)


def kernel(token_indices, table, pe):
    comb, gidx = pl.pallas_call(
        _prep_body,
        out_shape=(
            jax.ShapeDtypeStruct((SEQ, VOCAB, D), jnp.float32),
            jax.ShapeDtypeStruct((BATCH, SEQ), jnp.int32),
        ),
    )(token_indices, table, pe[:SEQ])

    comb = comb.reshape(TROWS, D)
    gidx3 = gidx.reshape(NW, NCHUNK, CHUNK)

    sc = pl.kernel(
        _sc_body,
        out_type=jax.ShapeDtypeStruct((ROWS, D), jnp.float32),
        mesh=plsc.VectorSubcoreMesh(
            core_axis_name="c", subcore_axis_name="s", num_cores=NC, num_subcores=NS
        ),
        scratch_types=[
            pltpu.VMEM((NCHUNK, CHUNK), jnp.int32),
            pltpu.VMEM((GROUP * CHUNK, D), jnp.float32),
            pltpu.VMEM((GROUP * CHUNK, D), jnp.float32),
            pltpu.VMEM_SHARED((TROWS, D), jnp.float32),
            pltpu.SemaphoreType.DMA,
            pltpu.SemaphoreType.DMA,
            pltpu.SemaphoreType.DMA,
            pltpu.SemaphoreType.DMA,
        ],
    )
    out = sc(comb, gidx3)
    return out.reshape(BATCH, SEQ, D)
